# Initial kernel scaffold; baseline (speedup 1.0000x reference)
import jax
import jax.numpy as jnp
from jax.experimental import pallas as pl


def kernel(inputs, table):
    raise NotImplementedError("write your pallas kernel here")


# speedup vs baseline: 1.0932x; 1.0932x over previous
import functools

import jax
import jax.numpy as jnp
from jax import lax
from jax.experimental import pallas as pl
from jax.experimental.pallas import tpu as pltpu
from jax.experimental.pallas import tpu_sc as plsc

NC = 2
NS = 16
NW = NC * NS

GRP = 128
K = 8
CHUNK = K * GRP


def _sc_gather(idx2d, table, B, D):
    rows_per_w = B // NW
    grp_per_w = rows_per_w // GRP
    n_chunks = grp_per_w // K
    mesh = plsc.VectorSubcoreMesh(core_axis_name="c", subcore_axis_name="s")

    @functools.partial(
        pl.kernel,
        out_type=jax.ShapeDtypeStruct((B, D), jnp.float32),
        mesh=mesh,
        scratch_types=[
            pltpu.VMEM((K, GRP), jnp.int32),
            pltpu.VMEM((CHUNK, D), jnp.float32),
            pltpu.SemaphoreType.DMA,
        ],
        compiler_params=pltpu.CompilerParams(use_tc_tiling_on_sc=False),
    )
    def k(table_hbm, idx_hbm, out_hbm, idx_v, rows_v, sem):
        wid = lax.axis_index("s") * NC + lax.axis_index("c")
        grp_base = wid * grp_per_w

        @pl.loop(0, n_chunks)
        def _chunk(i):
            g0 = pl.multiple_of(grp_base + i * K, 8)
            pltpu.sync_copy(idx_hbm.at[pl.ds(g0, K)], idx_v)
            copies = []
            for j in range(K):
                copies.append(
                    pltpu.async_copy(
                        table_hbm.at[idx_v.at[j]],
                        rows_v.at[pl.ds(j * GRP, GRP)],
                        sem,
                    )
                )
            for c in copies:
                c.wait()
            pltpu.sync_copy(rows_v, out_hbm.at[pl.ds(g0 * GRP, CHUNK)])

    return k(table, idx2d)


def kernel(inputs, table):
    B_, H = inputs.shape
    V, D = table.shape
    B = B_ * H
    idx2d = inputs.reshape(B // GRP, GRP)
    out = _sc_gather(idx2d, table, B, D)
    return out.reshape(B_, H, D)


# speedup vs baseline: 1.0979x; 1.0042x over previous
import functools

import jax
import jax.numpy as jnp
from jax import lax
from jax.experimental import pallas as pl
from jax.experimental.pallas import tpu as pltpu
from jax.experimental.pallas import tpu_sc as plsc

NC = 2
NS = 16
NW = NC * NS

CHUNK = 1280


def _sc_gather(idx, table, B, D):
    rows_per_w = B // NW
    n_chunks = rows_per_w // CHUNK
    mesh = plsc.VectorSubcoreMesh(core_axis_name="c", subcore_axis_name="s")

    @functools.partial(
        pl.kernel,
        out_type=jax.ShapeDtypeStruct((B, D), jnp.float32),
        mesh=mesh,
        scratch_types=[
            pltpu.VMEM((CHUNK,), jnp.int32),
            pltpu.VMEM((CHUNK, D), jnp.float32),
            pltpu.SemaphoreType.DMA,
        ],
        compiler_params=pltpu.CompilerParams(use_tc_tiling_on_sc=False),
    )
    def k(table_hbm, idx_hbm, out_hbm, idx_v, rows_v, sem):
        wid = lax.axis_index("s") * NC + lax.axis_index("c")
        row_base = wid * rows_per_w

        @pl.loop(0, n_chunks)
        def _chunk(i):
            r0 = pl.multiple_of(row_base + i * CHUNK, 8)
            pltpu.sync_copy(idx_hbm.at[pl.ds(r0, CHUNK)], idx_v)
            pltpu.async_copy(table_hbm.at[idx_v], rows_v, sem).wait()
            pltpu.sync_copy(rows_v, out_hbm.at[pl.ds(r0, CHUNK)])

    return k(table, idx)


def kernel(inputs, table):
    B_, H = inputs.shape
    V, D = table.shape
    B = B_ * H
    out = _sc_gather(inputs.reshape(B), table, B, D)
    return out.reshape(B_, H, D)


# speedup vs baseline: 1.7763x; 1.6180x over previous
import functools

import jax
import jax.numpy as jnp
from jax import lax
from jax.experimental import pallas as pl
from jax.experimental.pallas import tpu as pltpu
from jax.experimental.pallas import tpu_sc as plsc

NC = 2
NS = 16
NW = NC * NS

R = 32


def kernel(inputs, table):
    B, H = inputs.shape
    V, D = table.shape
    rows_per_w = B // NW
    n_chunks = rows_per_w // R
    mesh = plsc.VectorSubcoreMesh(core_axis_name="c", subcore_axis_name="s")

    @functools.partial(
        pl.kernel,
        out_type=jax.ShapeDtypeStruct((B, H, D), jnp.float32),
        mesh=mesh,
        scratch_types=[
            pltpu.VMEM((R, H), jnp.int32),
            pltpu.VMEM((R, H, D), jnp.float32),
            pltpu.SemaphoreType.DMA,
        ],
        compiler_params=pltpu.CompilerParams(use_tc_tiling_on_sc=False),
    )
    def k(table_hbm, idx_hbm, out_hbm, idx_v, rows_v, sem):
        wid = lax.axis_index("s") * NC + lax.axis_index("c")
        row_base = wid * rows_per_w

        @pl.loop(0, n_chunks)
        def _chunk(i):
            r0 = pl.multiple_of(row_base + i * R, 8)
            pltpu.sync_copy(idx_hbm.at[pl.ds(r0, R)], idx_v)

            @pl.loop(0, R)
            def _fire(r):
                pltpu.async_copy(table_hbm.at[idx_v.at[r]], rows_v.at[r], sem)

            pltpu.make_async_copy(out_hbm.at[pl.ds(r0, R)], rows_v, sem).wait()
            pltpu.sync_copy(rows_v, out_hbm.at[pl.ds(r0, R)])

    return k(table, inputs)
